# Initial kernel scaffold; baseline (speedup 1.0000x reference)
import jax
import jax.numpy as jnp
from jax.experimental import pallas as pl


def kernel(x, edge_index, W1, b1, W2, b2, W3, b3, W4, b4, W5, b5, W6, b6):
    raise NotImplementedError("write your pallas kernel here")


# speedup vs baseline: 6.7496x; 6.7496x over previous
import functools

import jax
import jax.numpy as jnp
from jax import lax
from jax.experimental import pallas as pl
from jax.experimental.pallas import tpu as pltpu
from jax.experimental.pallas import tpu_sc as plsc

N_MOV = 10000
N_PHASE = 10000
E = 320000
D = 128
H = 128

NC = 2
NS = 16
EPT = E // (NC * NS)
K = 80
NCH = EPT // K
ACC = 10240
RPT = ACC // NS
ZR = 16
CW = 128

_ROW_BLK = 2000


def _mlp3_body(x_ref, w1, b1, w2, b2, w3, b3, o_ref):
    h = jnp.dot(x_ref[...], w1[...], preferred_element_type=jnp.float32)
    h = jnp.maximum(h + b1[...], 0.0)
    h = jnp.dot(h, w2[...], preferred_element_type=jnp.float32)
    h = jnp.maximum(h + b2[...], 0.0)
    h = jnp.dot(h, w3[...], preferred_element_type=jnp.float32)
    h = jnp.maximum(h + b3[...], 0.0)
    o_ref[...] = h


def _mlp3(x, W1, b1, W2, b2, W3, b3):
    nblk = N_MOV // _ROW_BLK
    full = lambda shape: pl.BlockSpec(shape, lambda i: (0, 0))
    return pl.pallas_call(
        _mlp3_body,
        grid=(nblk,),
        in_specs=[
            pl.BlockSpec((_ROW_BLK, D), lambda i: (i, 0)),
            full((D, H)), full((1, H)),
            full((H, H)), full((1, H)),
            full((H, H)), full((1, H)),
        ],
        out_specs=pl.BlockSpec((_ROW_BLK, H), lambda i: (i, 0)),
        out_shape=jax.ShapeDtypeStruct((N_MOV, H), jnp.float32),
    )(x, W1, b1, W2, b2, W3, b3)


def _sum_body(h_hbm, src_hbm, dst_hbm, sums_hbm,
              src_v, dst_v, cdst_v, rows_v, zacc_v, acc_sh, gsem):
    cid = lax.axis_index("c")
    sid = lax.axis_index("s")
    w = cid * NS + sid

    def _fill_zacc(t, carry):
        zacc_v[t // 8, pl.ds((t % 8) * 16, 16)] = jnp.zeros((16,), jnp.float32)
        return carry
    lax.fori_loop(0, ZR * 8, _fill_zacc, 0)

    r0 = sid * RPT

    def _zero(t, carry):
        pltpu.sync_copy(zacc_v, acc_sh.at[pl.ds(r0 + t * ZR, ZR)])
        return carry
    lax.fori_loop(0, RPT // ZR, _zero, 0)

    pltpu.sync_copy(src_hbm.at[w], src_v)
    pltpu.sync_copy(dst_hbm.at[w], dst_v)

    plsc.subcore_barrier()

    def _step(j, carry):
        for l in range(K // 16):
            cdst_v[0, pl.ds(l * 16, 16)] = dst_v[pl.ds(j * K + l * 16, 16)]
        pltpu.async_copy(h_hbm.at[src_v.at[pl.ds(j * K, K)]],
                         rows_v, gsem).wait()
        pltpu.sync_copy(rows_v, acc_sh.at[cdst_v.at[0]], add=True)
        return carry
    lax.fori_loop(0, NCH, _step, 0)

    plsc.subcore_barrier()

    g0 = cid * ACC + r0
    pltpu.sync_copy(acc_sh.at[pl.ds(r0, RPT)], sums_hbm.at[pl.ds(g0, RPT)])


@functools.partial(
    pl.kernel,
    out_type=[jax.ShapeDtypeStruct((NC * ACC, D), jnp.float32)],
    mesh=plsc.VectorSubcoreMesh(
        core_axis_name="c", subcore_axis_name="s",
        num_cores=NC, num_subcores=NS),
    scratch_types=[
        pltpu.VMEM((EPT,), jnp.int32),
        pltpu.VMEM((EPT,), jnp.int32),
        pltpu.VMEM((1, K), jnp.int32),
        pltpu.VMEM((K, D), jnp.float32),
        pltpu.VMEM((ZR, D), jnp.float32),
        pltpu.VMEM_SHARED((ACC, D), jnp.float32),
        pltpu.SemaphoreType.DMA,
    ],
)
def _agg_sums(h_hbm, src_hbm, dst_hbm, sums_hbm, *scratch):
    _sum_body(h_hbm, src_hbm, dst_hbm, sums_hbm, *scratch)


def _cnt_body(dst_hbm, cnts_hbm, dst_v, cdst_v, ones_v, zcnt_v, cnt_sh):
    cid = lax.axis_index("c")
    sid = lax.axis_index("s")
    w = cid * NS + sid

    def _fill_ones(t, carry):
        ones_v[t // 8, pl.ds((t % 8) * 16, 16)] = jnp.ones((16,), jnp.float32)
        return carry
    lax.fori_loop(0, K * 8, _fill_ones, 0)

    def _fill_zcnt(t, carry):
        zcnt_v[t // 8, pl.ds((t % 8) * 16, 16)] = jnp.zeros((16,), jnp.float32)
        return carry
    lax.fori_loop(0, ZR * 8, _fill_zcnt, 0)

    r0 = sid * RPT

    def _zero(t, carry):
        pltpu.sync_copy(zcnt_v, cnt_sh.at[pl.ds(r0 + t * ZR, ZR)])
        return carry
    lax.fori_loop(0, RPT // ZR, _zero, 0)

    pltpu.sync_copy(dst_hbm.at[w], dst_v)

    plsc.subcore_barrier()

    def _step(j, carry):
        for l in range(K // 16):
            cdst_v[0, pl.ds(l * 16, 16)] = dst_v[pl.ds(j * K + l * 16, 16)]
        pltpu.sync_copy(ones_v, cnt_sh.at[cdst_v.at[0]], add=True)
        return carry
    lax.fori_loop(0, NCH, _step, 0)

    plsc.subcore_barrier()

    g0 = cid * ACC + r0
    pltpu.sync_copy(cnt_sh.at[pl.ds(r0, RPT)], cnts_hbm.at[pl.ds(g0, RPT)])


@functools.partial(
    pl.kernel,
    out_type=[jax.ShapeDtypeStruct((NC * ACC, CW), jnp.float32)],
    mesh=plsc.VectorSubcoreMesh(
        core_axis_name="c", subcore_axis_name="s",
        num_cores=NC, num_subcores=NS),
    scratch_types=[
        pltpu.VMEM((EPT,), jnp.int32),
        pltpu.VMEM((1, K), jnp.int32),
        pltpu.VMEM((K, CW), jnp.float32),
        pltpu.VMEM((ZR, CW), jnp.float32),
        pltpu.VMEM_SHARED((ACC, CW), jnp.float32),
    ],
)
def _agg_cnts(dst_hbm, cnts_hbm, *scratch):
    _cnt_body(dst_hbm, cnts_hbm, *scratch)


def _mlp2_body(s0_ref, s1_ref, c0_ref, c1_ref, w4, b4, w5, b5, w6, b6, o_ref):
    s = s0_ref[0] + s1_ref[0]
    c = c0_ref[0] + c1_ref[0]
    cnt = jnp.max(c, axis=1, keepdims=True)
    agg = s / jnp.maximum(cnt, 1.0)
    g = jnp.dot(agg, w4[...], preferred_element_type=jnp.float32)
    g = jnp.maximum(g + b4[...], 0.0)
    g = jnp.dot(g, w5[...], preferred_element_type=jnp.float32)
    g = jnp.maximum(g + b5[...], 0.0)
    o_ref[...] = jnp.dot(g, w6[...], preferred_element_type=jnp.float32) + b6[...]


def _mlp2(sums, cnts, W4, b4, W5, b5, W6, b6):
    nblk = N_PHASE // _ROW_BLK
    full = lambda shape: pl.BlockSpec(shape, lambda i: (0, 0))
    return pl.pallas_call(
        _mlp2_body,
        grid=(nblk,),
        in_specs=[
            pl.BlockSpec((1, _ROW_BLK, D), lambda i: (0, i, 0)),
            pl.BlockSpec((1, _ROW_BLK, D), lambda i: (1, i, 0)),
            pl.BlockSpec((1, _ROW_BLK, CW), lambda i: (0, i, 0)),
            pl.BlockSpec((1, _ROW_BLK, CW), lambda i: (1, i, 0)),
            full((H, H)), full((1, H)),
            full((H, H)), full((1, H)),
            full((H, 1)), full((1, 1)),
        ],
        out_specs=pl.BlockSpec((_ROW_BLK, 1), lambda i: (i, 0)),
        out_shape=jax.ShapeDtypeStruct((N_PHASE, 1), jnp.float32),
    )(sums, sums, cnts, cnts, W4, b4, W5, b5, W6, b6)


def _unwrap(res):
    return res[0] if isinstance(res, (list, tuple)) else res


def kernel(x, edge_index, W1, b1, W2, b2, W3, b3, W4, b4, W5, b5, W6, b6):
    h = _mlp3(x, W1, b1.reshape(1, H), W2, b2.reshape(1, H),
              W3, b3.reshape(1, H))
    src = edge_index[0].reshape(NC * NS, EPT)
    dst = edge_index[1].reshape(NC * NS, EPT)
    sums = _unwrap(_agg_sums(h, src, dst)).reshape(NC, ACC, D)
    cnts = _unwrap(_agg_cnts(dst)).reshape(NC, ACC, CW)
    out = _mlp2(sums, cnts, W4, b4.reshape(1, H), W5, b5.reshape(1, H),
                W6, b6.reshape(1, 1))
    return out[:, 0]


# speedup vs baseline: 9.1627x; 1.3575x over previous
import functools

import jax
import jax.numpy as jnp
from jax import lax
from jax.experimental import pallas as pl
from jax.experimental.pallas import tpu as pltpu
from jax.experimental.pallas import tpu_sc as plsc

N_MOV = 10000
N_PHASE = 10000
E = 320000
D = 128
H = 128

NC = 2
NS = 16
EPT = E // (NC * NS)
K = 80
NCH = EPT // K
ACC = 10240
RPT = ACC // NS
ZR = 16
CW = 128

_ROW_BLK = 2000


def _mlp3_body(x_ref, w1, b1, w2, b2, w3, b3, o_ref):
    h = jnp.dot(x_ref[...], w1[...], preferred_element_type=jnp.float32)
    h = jnp.maximum(h + b1[...], 0.0)
    h = jnp.dot(h, w2[...], preferred_element_type=jnp.float32)
    h = jnp.maximum(h + b2[...], 0.0)
    h = jnp.dot(h, w3[...], preferred_element_type=jnp.float32)
    h = jnp.maximum(h + b3[...], 0.0)
    o_ref[...] = h


def _mlp3(x, W1, b1, W2, b2, W3, b3):
    nblk = N_MOV // _ROW_BLK
    full = lambda shape: pl.BlockSpec(shape, lambda i: (0, 0))
    return pl.pallas_call(
        _mlp3_body,
        grid=(nblk,),
        in_specs=[
            pl.BlockSpec((_ROW_BLK, D), lambda i: (i, 0)),
            full((D, H)), full((1, H)),
            full((H, H)), full((1, H)),
            full((H, H)), full((1, H)),
        ],
        out_specs=pl.BlockSpec((_ROW_BLK, H), lambda i: (i, 0)),
        out_shape=jax.ShapeDtypeStruct((N_MOV, H), jnp.float32),
    )(x, W1, b1, W2, b2, W3, b3)


def _sum_body(h_hbm, src_hbm, dst_hbm, sums_hbm,
              src_v, dst_v, cdst_v, rows0_v, rows1_v, zacc_v, acc_sh,
              gsem0, gsem1):
    cid = lax.axis_index("c")
    sid = lax.axis_index("s")
    w = cid * NS + sid

    def _fill_zacc(t, carry):
        zacc_v[t // 8, pl.ds((t % 8) * 16, 16)] = jnp.zeros((16,), jnp.float32)
        return carry
    lax.fori_loop(0, ZR * 8, _fill_zacc, 0)

    r0 = sid * RPT

    def _zero(t, carry):
        pltpu.sync_copy(zacc_v, acc_sh.at[pl.ds(r0 + t * ZR, ZR)])
        return carry
    lax.fori_loop(0, RPT // ZR, _zero, 0)

    pltpu.sync_copy(src_hbm.at[w], src_v)
    pltpu.sync_copy(dst_hbm.at[w], dst_v)

    plsc.subcore_barrier()

    def _issue(j, buf, sem):
        pltpu.async_copy(h_hbm.at[src_v.at[pl.ds(j * K, K)]], buf, sem)

    def _wait(j, buf, sem):
        pltpu.make_async_copy(
            h_hbm.at[src_v.at[pl.ds(j * K, K)]], buf, sem).wait()

    def _scat(j, buf):
        for l in range(K // 16):
            cdst_v[0, pl.ds(l * 16, 16)] = dst_v[pl.ds(j * K + l * 16, 16)]
        pltpu.sync_copy(buf, acc_sh.at[cdst_v.at[0]], add=True)

    _issue(0, rows0_v, gsem0)

    def _pair(p, carry):
        j0 = 2 * p
        _issue(j0 + 1, rows1_v, gsem1)
        _wait(j0, rows0_v, gsem0)
        _scat(j0, rows0_v)
        _issue(j0 + 2, rows0_v, gsem0)
        _wait(j0 + 1, rows1_v, gsem1)
        _scat(j0 + 1, rows1_v)
        return carry
    lax.fori_loop(0, NCH // 2, _pair, 0)

    _wait(NCH - 1, rows0_v, gsem0)
    _scat(NCH - 1, rows0_v)

    plsc.subcore_barrier()

    g0 = cid * ACC + r0
    pltpu.sync_copy(acc_sh.at[pl.ds(r0, RPT)], sums_hbm.at[pl.ds(g0, RPT)])


@functools.partial(
    pl.kernel,
    out_type=[jax.ShapeDtypeStruct((NC * ACC, D), jnp.float32)],
    mesh=plsc.VectorSubcoreMesh(
        core_axis_name="c", subcore_axis_name="s",
        num_cores=NC, num_subcores=NS),
    scratch_types=[
        pltpu.VMEM((EPT,), jnp.int32),
        pltpu.VMEM((EPT,), jnp.int32),
        pltpu.VMEM((1, K), jnp.int32),
        pltpu.VMEM((K, D), jnp.float32),
        pltpu.VMEM((K, D), jnp.float32),
        pltpu.VMEM((ZR, D), jnp.float32),
        pltpu.VMEM_SHARED((ACC, D), jnp.float32),
        pltpu.SemaphoreType.DMA,
        pltpu.SemaphoreType.DMA,
    ],
)
def _agg_sums(h_hbm, src_hbm, dst_hbm, sums_hbm, *scratch):
    _sum_body(h_hbm, src_hbm, dst_hbm, sums_hbm, *scratch)


def _cnt_body(dst_hbm, cnts_hbm, dst_v, cdst_v, ones_v, zcnt_v, cnt_sh):
    cid = lax.axis_index("c")
    sid = lax.axis_index("s")
    w = cid * NS + sid

    def _fill_ones(t, carry):
        ones_v[t // 8, pl.ds((t % 8) * 16, 16)] = jnp.ones((16,), jnp.float32)
        return carry
    lax.fori_loop(0, K * 8, _fill_ones, 0)

    def _fill_zcnt(t, carry):
        zcnt_v[t // 8, pl.ds((t % 8) * 16, 16)] = jnp.zeros((16,), jnp.float32)
        return carry
    lax.fori_loop(0, ZR * 8, _fill_zcnt, 0)

    r0 = sid * RPT

    def _zero(t, carry):
        pltpu.sync_copy(zcnt_v, cnt_sh.at[pl.ds(r0 + t * ZR, ZR)])
        return carry
    lax.fori_loop(0, RPT // ZR, _zero, 0)

    pltpu.sync_copy(dst_hbm.at[w], dst_v)

    plsc.subcore_barrier()

    def _step(j, carry):
        for l in range(K // 16):
            cdst_v[0, pl.ds(l * 16, 16)] = dst_v[pl.ds(j * K + l * 16, 16)]
        pltpu.sync_copy(ones_v, cnt_sh.at[cdst_v.at[0]], add=True)
        return carry
    lax.fori_loop(0, NCH, _step, 0)

    plsc.subcore_barrier()

    g0 = cid * ACC + r0
    pltpu.sync_copy(cnt_sh.at[pl.ds(r0, RPT)], cnts_hbm.at[pl.ds(g0, RPT)])


@functools.partial(
    pl.kernel,
    out_type=[jax.ShapeDtypeStruct((NC * ACC, CW), jnp.float32)],
    mesh=plsc.VectorSubcoreMesh(
        core_axis_name="c", subcore_axis_name="s",
        num_cores=NC, num_subcores=NS),
    scratch_types=[
        pltpu.VMEM((EPT,), jnp.int32),
        pltpu.VMEM((1, K), jnp.int32),
        pltpu.VMEM((K, CW), jnp.float32),
        pltpu.VMEM((ZR, CW), jnp.float32),
        pltpu.VMEM_SHARED((ACC, CW), jnp.float32),
    ],
)
def _agg_cnts(dst_hbm, cnts_hbm, *scratch):
    _cnt_body(dst_hbm, cnts_hbm, *scratch)


def _mlp2_body(s0_ref, s1_ref, c0_ref, c1_ref, w4, b4, w5, b5, w6, b6, o_ref):
    s = s0_ref[0] + s1_ref[0]
    c = c0_ref[0] + c1_ref[0]
    cnt = jnp.max(c, axis=1, keepdims=True)
    agg = s / jnp.maximum(cnt, 1.0)
    g = jnp.dot(agg, w4[...], preferred_element_type=jnp.float32)
    g = jnp.maximum(g + b4[...], 0.0)
    g = jnp.dot(g, w5[...], preferred_element_type=jnp.float32)
    g = jnp.maximum(g + b5[...], 0.0)
    o_ref[...] = jnp.dot(g, w6[...], preferred_element_type=jnp.float32) + b6[...]


def _mlp2(sums, cnts, W4, b4, W5, b5, W6, b6):
    nblk = N_PHASE // _ROW_BLK
    full = lambda shape: pl.BlockSpec(shape, lambda i: (0, 0))
    return pl.pallas_call(
        _mlp2_body,
        grid=(nblk,),
        in_specs=[
            pl.BlockSpec((1, _ROW_BLK, D), lambda i: (0, i, 0)),
            pl.BlockSpec((1, _ROW_BLK, D), lambda i: (1, i, 0)),
            pl.BlockSpec((1, _ROW_BLK, CW), lambda i: (0, i, 0)),
            pl.BlockSpec((1, _ROW_BLK, CW), lambda i: (1, i, 0)),
            full((H, H)), full((1, H)),
            full((H, H)), full((1, H)),
            full((H, 1)), full((1, 1)),
        ],
        out_specs=pl.BlockSpec((_ROW_BLK, 1), lambda i: (i, 0)),
        out_shape=jax.ShapeDtypeStruct((N_PHASE, 1), jnp.float32),
    )(sums, sums, cnts, cnts, W4, b4, W5, b5, W6, b6)


def _unwrap(res):
    return res[0] if isinstance(res, (list, tuple)) else res


def kernel(x, edge_index, W1, b1, W2, b2, W3, b3, W4, b4, W5, b5, W6, b6):
    h = _mlp3(x, W1, b1.reshape(1, H), W2, b2.reshape(1, H),
              W3, b3.reshape(1, H))
    src = edge_index[0].reshape(NC * NS, EPT)
    dst = edge_index[1].reshape(NC * NS, EPT)
    sums = _unwrap(_agg_sums(h, src, dst)).reshape(NC, ACC, D)
    cnts = _unwrap(_agg_cnts(dst)).reshape(NC, ACC, CW)
    out = _mlp2(sums, cnts, W4, b4.reshape(1, H), W5, b5.reshape(1, H),
                W6, b6.reshape(1, 1))
    return out[:, 0]
